# baseline (speedup 1.0000x reference)
import functools

import jax
import jax.numpy as jnp
from jax import lax
from jax.experimental import pallas as pl
from jax.experimental.pallas import tpu as pltpu
from jax.experimental.pallas import tpu_sc as plsc

B = 16384
D = 16
CAP = 100000
INV_GRID = 100000.0
RPB = 8
NBLK = CAP // RPB

_info = plsc.get_sparse_core_info()
_NC, _NS, _L = _info.num_cores, _info.num_subcores, _info.num_lanes
NW = _NC * _NS
BPW = B // NW
CHUNK = 16
NCH = BPW // CHUNK
NGRP = CHUNK // 16


@functools.partial(
    pl.kernel,
    out_type=jax.ShapeDtypeStruct((B, 2 * D), jnp.float32),
    mesh=plsc.VectorSubcoreMesh(core_axis_name="c", subcore_axis_name="s"),
    compiler_params=pltpu.CompilerParams(needs_layout_passes=False),
    scratch_types=[
        pltpu.VMEM((2, BPW), jnp.float32),
        pltpu.VMEM((2, BPW), jnp.int32),
        pltpu.VMEM((2, NCH, CHUNK), jnp.int32),
        pltpu.SMEM((2, BPW), jnp.int32),
        pltpu.VMEM_SHARED((_NS, 2, BPW), jnp.int32),
        pltpu.VMEM((CHUNK * RPB, D), jnp.float32),
        pltpu.VMEM((CHUNK * RPB, D), jnp.float32),
        pltpu.VMEM((CHUNK * RPB, D), jnp.float32),
        pltpu.VMEM((CHUNK * RPB, D), jnp.float32),
        pltpu.VMEM((2, CHUNK, 2 * D), jnp.float32),
        pltpu.SemaphoreType.DMA,
        pltpu.SemaphoreType.DMA,
        pltpu.SemaphoreType.DMA,
        pltpu.SemaphoreType.DMA,
    ],
)
def _grid_gather(obs_t, t0b, t1b, out, obs_v, idx_v, sub_v, idx_s, idx_h,
                 blk00, blk01, blk10, blk11, o_v, isem, gsem0, gsem1, osem):
    wid = lax.axis_index("s") * _NC + lax.axis_index("c")
    base = wid * BPW
    pltpu.sync_copy(obs_t.at[0, pl.ds(base, BPW)], obs_v.at[0])
    pltpu.sync_copy(obs_t.at[1, pl.ds(base, BPW)], obs_v.at[1])
    for f in range(2):
        for c in range(NCH):
            for j in range(CHUNK // _L):
                x = obs_v[f, pl.ds(c * CHUNK + j * _L, _L)]
                r = (x * INV_GRID).astype(jnp.int32)
                r = jnp.minimum(jnp.maximum(r, 0), CAP - 1)
                idx_v[f, pl.ds(c * CHUNK + j * _L, _L)] = (r >> 3) << 3
                sub_v[f, c, pl.ds(j * _L, _L)] = r & 7
    sid = lax.axis_index("s")
    pltpu.async_copy(idx_v, idx_h.at[sid], isem).wait()
    pltpu.sync_copy(idx_h.at[sid], idx_s)
    slots = ((blk00, blk01), (blk10, blk11))

    gsems = (gsem0, gsem1)

    def fire(c, slot):
        b0_v, b1_v = slots[slot]
        gsem = gsems[slot]

        def body(j, carry):
            b0 = pl.multiple_of(idx_s[0, c * CHUNK + j], RPB)
            pltpu.async_copy(t0b.at[pl.ds(b0, RPB)],
                             b0_v.at[pl.ds(j * RPB, RPB)], gsem)
            b1 = pl.multiple_of(idx_s[1, c * CHUNK + j], RPB)
            pltpu.async_copy(t1b.at[pl.ds(b1, RPB)],
                             b1_v.at[pl.ds(j * RPB, RPB)], gsem)
            return carry

        lax.fori_loop(0, CHUNK, body, 0)

    def drain(slot):
        b0_v, b1_v = slots[slot]
        gsem = gsems[slot]
        pltpu.make_async_copy(t0b.at[pl.ds(0, CHUNK * RPB)], b0_v,
                              gsem).wait()
        pltpu.make_async_copy(t1b.at[pl.ds(0, CHUNK * RPB)], b1_v,
                              gsem).wait()

    fire(0, 0)
    rows = lax.iota(jnp.int32, _L)
    for c in range(NCH):
        slot = c % 2
        if c + 1 < NCH:
            fire(c + 1, 1 - slot)
        drain(slot)
        for f in range(2):
            blk = slots[slot][f]
            for g in range(NGRP):
                grows = rows + g * 16
                subs = sub_v[f, c, pl.ds(g * 16, 16)]
                brows = (grows << 3) + subs
                for e in range(D):
                    vals = plsc.load_gather(
                        blk, [brows, jnp.full((16,), e, jnp.int32)])
                    plsc.store_scatter(o_v.at[slot], [grows,
                                       jnp.full((16,), f * D + e, jnp.int32)],
                                       vals)
        pltpu.async_copy(o_v.at[slot],
                         out.at[pl.ds(base + c * CHUNK, CHUNK)], osem).wait()


def kernel(obs, table0, table1):
    obs_t = obs.T
    return _grid_gather(obs_t, table0, table1)
